# baseline (speedup 1.0000x reference)
import functools

import jax
import jax.numpy as jnp
from jax import lax
from jax.experimental import pallas as pl
from jax.experimental.pallas import tpu as pltpu
from jax.experimental.pallas import tpu_sc as plsc

_IMASK = -2048
_IMAX = 2147483647


def _linear_bn_relu_body(n, eps, x_ref, wt_ref, b_ref, g_ref, be_ref,
                         z_ref, y_s, acc_s):
    p = pl.program_id(0)
    j = pl.program_id(1)
    tile = x_ref.shape[0]

    @pl.when((p == 0) & (j == 0))
    def _():
        acc_s[...] = jnp.zeros_like(acc_s)

    @pl.when(p == 0)
    def _():
        y = lax.dot_general(
            x_ref[...], wt_ref[...], (((1,), (1,)), ((), ())),
            preferred_element_type=jnp.float32) + b_ref[...]
        y_s[pl.ds(j * tile, tile), :] = y
        s = jnp.sum(y, axis=0, keepdims=True)
        ss = jnp.sum(y * y, axis=0, keepdims=True)
        acc_s[...] += jnp.concatenate([s, ss], axis=0)

    @pl.when(p == 1)
    def _():
        mean = acc_s[0:1, :] / n
        var = acc_s[1:2, :] / n - mean * mean
        sc = g_ref[...] * lax.rsqrt(var + eps)
        sh = be_ref[...] - mean * sc
        z_ref[...] = jnp.maximum(
            y_s[pl.ds(j * tile, tile), :] * sc + sh, 0.0)


def _linear_bn_relu(x, w, b, gamma, beta, tile, eps=1e-5):
    n, _ = x.shape
    c = w.shape[0]
    body = functools.partial(_linear_bn_relu_body, float(n), eps)
    return pl.pallas_call(
        body,
        grid=(2, n // tile),
        in_specs=[
            pl.BlockSpec((tile, x.shape[1]), lambda p, j: ((1 - p) * j, 0)),
            pl.BlockSpec(w.shape, lambda p, j: (0, 0)),
            pl.BlockSpec((1, c), lambda p, j: (0, 0)),
            pl.BlockSpec((1, c), lambda p, j: (0, 0)),
            pl.BlockSpec((1, c), lambda p, j: (0, 0)),
        ],
        out_specs=pl.BlockSpec((tile, c), lambda p, j: (p * j, 0)),
        out_shape=jax.ShapeDtypeStruct((n, c), jnp.float32),
        scratch_shapes=[
            pltpu.VMEM((n, c), jnp.float32),
            pltpu.VMEM((2, c), jnp.float32),
        ],
    )(x, w, b[None, :], gamma[None, :], beta[None, :])


def _knn_topk_body(p1_ref, p2_ref, i1_ref, i2_ref, i3_ref, wexp_ref):
    at = jnp.transpose(p1_ref[0], (1, 0))
    pk = p2_ref[0]
    tq, n2 = at.shape[1], pk.shape[0]
    acc = jnp.zeros((n2, tq), jnp.float32)
    for d in range(3):
        t = pk[:, d:d + 1] - at[d:d + 1, :]
        acc = acc + t * t

    sub = lax.broadcasted_iota(jnp.int32, (n2, tq), 0)
    key = (lax.bitcast_convert_type(acc, jnp.int32) & _IMASK) | sub
    k1 = jnp.min(key, axis=0, keepdims=True)
    keyb = jnp.where(key == k1, _IMAX, key)
    k2 = jnp.min(keyb, axis=0, keepdims=True)
    keyc = jnp.where(keyb == k2, _IMAX, keyb)
    k3 = jnp.min(keyc, axis=0, keepdims=True)

    i1, i2, i3 = k1 & 2047, k2 & 2047, k3 & 2047
    d1 = lax.bitcast_convert_type(k1 - i1, jnp.float32)
    d2 = lax.bitcast_convert_type(k2 - i2, jnp.float32)
    d3 = lax.bitcast_convert_type(k3 - i3, jnp.float32)
    r1 = 1.0 / (d1 + 1e-8)
    r2 = 1.0 / (d2 + 1e-8)
    r3 = 1.0 / (d3 + 1e-8)
    inv = 1.0 / (r1 + r2 + r3)
    boff = pl.program_id(0) * n2
    i1_ref[0, 0] = i1 + boff
    i2_ref[0, 0] = i2 + boff
    i3_ref[0, 0] = i3 + boff
    w1 = jnp.transpose(r1 * inv, (1, 0))
    w2 = jnp.transpose(r2 * inv, (1, 0))
    w3 = jnp.transpose(r3 * inv, (1, 0))
    wexp_ref[0, 0] = jnp.concatenate(
        [jnp.broadcast_to(w1, (tq, 16)),
         jnp.broadcast_to(w2, (tq, 16)),
         jnp.broadcast_to(w3, (tq, 16))], axis=1)


def _knn_topk(p1, p2, tile):
    b, n1, _ = p1.shape
    n2 = p2.shape[1]
    nt = n1 // tile
    ispec = pl.BlockSpec((1, 1, 1, tile), lambda i, j: (i, j, 0, 0))
    ishape = jax.ShapeDtypeStruct((b, nt, 1, tile), jnp.int32)
    return pl.pallas_call(
        _knn_topk_body,
        grid=(b, nt),
        in_specs=[
            pl.BlockSpec((1, tile, 3), lambda i, j: (i, j, 0)),
            pl.BlockSpec((1, n2, 3), lambda i, j: (i, 0, 0)),
        ],
        out_specs=[
            ispec, ispec, ispec,
            pl.BlockSpec((1, 1, tile, 48), lambda i, j: (i, j, 0, 0)),
        ],
        out_shape=[
            ishape, ishape, ishape,
            jax.ShapeDtypeStruct((b, nt, tile, 48), jnp.float32),
        ],
    )(p1, p2)


def _make_sc_interp(n_rows, c, n_workers, chunk):
    qpw = n_rows // n_workers
    n_chunks = qpw // chunk
    mesh = plsc.VectorSubcoreMesh(core_axis_name="c", subcore_axis_name="s")

    @functools.partial(
        pl.kernel, mesh=mesh,
        out_type=jax.ShapeDtypeStruct((n_rows, c), jnp.float32),
        scratch_types=[
            pltpu.VMEM((3, chunk), jnp.int32),
            pltpu.VMEM((3, chunk), jnp.int32),
            pltpu.VMEM((3, chunk), jnp.int32),
            pltpu.VMEM((3, chunk, 48), jnp.float32),
            pltpu.VMEM((3, chunk, c), jnp.float32),
            pltpu.VMEM((2, chunk, c), jnp.float32),
            pltpu.VMEM((2, chunk, c), jnp.float32),
            pltpu.VMEM((2, chunk, c), jnp.float32),
            pltpu.VMEM((2, chunk, c), jnp.float32),
        ] + [pltpu.SemaphoreType.DMA] * 7,
    )
    def sc_interp(z2_hbm, x1p_hbm, i1_hbm, i2_hbm, i3_hbm, wexp_hbm, out_hbm,
                  i1_v, i2_v, i3_v, w_v, x1p_v,
                  r1_v, r2_v, r3_v, out_v,
                  sl0, sl1, sl2, sg0, sg1, ss0, ss1):
        wid = lax.axis_index("s") * 2 + lax.axis_index("c")
        wbase = wid * qpw
        sls = (sl0, sl1, sl2)
        sgs = (sg0, sg1)
        sss = (ss0, ss1)
        lin_h = {}
        g_h = {}
        st_h = {}

        def issue_linear(ci):
            p = ci % 3
            sl = pl.ds(wbase + ci * chunk, chunk)
            lin_h[ci] = [
                pltpu.async_copy(i1_hbm.at[sl], i1_v.at[p], sls[p]),
                pltpu.async_copy(i2_hbm.at[sl], i2_v.at[p], sls[p]),
                pltpu.async_copy(i3_hbm.at[sl], i3_v.at[p], sls[p]),
                pltpu.async_copy(wexp_hbm.at[sl], w_v.at[p], sls[p]),
                pltpu.async_copy(x1p_hbm.at[sl], x1p_v.at[p], sls[p]),
            ]

        def issue_gathers(ci):
            p3, p2 = ci % 3, ci % 2
            for h in lin_h.pop(ci):
                h.wait()
            g_h[ci] = [
                pltpu.async_copy(z2_hbm.at[i1_v.at[p3]], r1_v.at[p2], sgs[p2]),
                pltpu.async_copy(z2_hbm.at[i2_v.at[p3]], r2_v.at[p2], sgs[p2]),
                pltpu.async_copy(z2_hbm.at[i3_v.at[p3]], r3_v.at[p2], sgs[p2]),
            ]

        issue_linear(0)
        issue_gathers(0)
        if n_chunks > 1:
            issue_linear(1)

        for ci in range(n_chunks):
            if ci + 2 < n_chunks:
                issue_linear(ci + 2)
            if ci + 1 < n_chunks:
                issue_gathers(ci + 1)
            p3, p2 = ci % 3, ci % 2
            for h in g_h.pop(ci):
                h.wait()
            if ci - 2 in st_h:
                st_h.pop(ci - 2).wait()

            def q_body(q, _):
                a1 = w_v[p3, q, 0:16]
                a2 = w_v[p3, q, 16:32]
                a3 = w_v[p3, q, 32:48]
                for cc in range(c // 16):
                    s = pl.ds(cc * 16, 16)
                    out_v[p2, q, s] = (
                        a1 * r1_v[p2, q, s] + a2 * r2_v[p2, q, s]
                        + a3 * r3_v[p2, q, s] + x1p_v[p3, q, s])
                return 0

            lax.fori_loop(0, chunk, q_body, 0)
            st_h[ci] = pltpu.async_copy(
                out_v.at[p2], out_hbm.at[pl.ds(wbase + ci * chunk, chunk)],
                sss[p2])
        for h in st_h.values():
            h.wait()

    return sc_interp


@jax.jit
def kernel(p1, x1, p2, x2, W1, b1, g1, be1, W2, b2, g2, be2):
    B, N1, _ = p1.shape
    N2 = p2.shape[1]
    C = W1.shape[0]

    z2 = _linear_bn_relu(x2.reshape(B * N2, -1), W2, b2, g2, be2, 1024)
    x1p = _linear_bn_relu(x1.reshape(B * N1, -1), W1, b1, g1, be1, 1024)

    tile = 512
    i1, i2, i3, wexp = _knn_topk(p1, p2, tile)
    sc_interp = _make_sc_interp(B * N1, C, 32, 64)
    out = sc_interp(z2, x1p, i1.reshape(B * N1), i2.reshape(B * N1),
                    i3.reshape(B * N1), wexp.reshape(B * N1, 48))
    return out.reshape(B, N1, C)


# speedup vs baseline: 1.0994x; 1.0994x over previous
import functools

import jax
import jax.numpy as jnp
from jax import lax
from jax.experimental import pallas as pl
from jax.experimental.pallas import tpu as pltpu
from jax.experimental.pallas import tpu_sc as plsc

_IMASK = -2048
_IMAX = 2147483647


def _linear_bn_relu_body(n, eps, x_ref, wt_ref, b_ref, g_ref, be_ref,
                         z_ref, y_s, acc_s):
    p = pl.program_id(0)
    j = pl.program_id(1)
    tile = x_ref.shape[0]

    @pl.when((p == 0) & (j == 0))
    def _():
        acc_s[...] = jnp.zeros_like(acc_s)

    @pl.when(p == 0)
    def _():
        y = lax.dot_general(
            x_ref[...], wt_ref[...], (((1,), (1,)), ((), ())),
            preferred_element_type=jnp.float32) + b_ref[...]
        y_s[pl.ds(j * tile, tile), :] = y
        s = jnp.sum(y, axis=0, keepdims=True)
        ss = jnp.sum(y * y, axis=0, keepdims=True)
        acc_s[...] += jnp.concatenate([s, ss], axis=0)

    @pl.when(p == 1)
    def _():
        mean = acc_s[0:1, :] / n
        var = acc_s[1:2, :] / n - mean * mean
        sc = g_ref[...] * lax.rsqrt(var + eps)
        sh = be_ref[...] - mean * sc
        z_ref[...] = jnp.maximum(
            y_s[pl.ds(j * tile, tile), :] * sc + sh, 0.0)


def _linear_bn_relu(x, w, b, gamma, beta, tile, eps=1e-5):
    n, _ = x.shape
    c = w.shape[0]
    body = functools.partial(_linear_bn_relu_body, float(n), eps)
    return pl.pallas_call(
        body,
        grid=(2, n // tile),
        in_specs=[
            pl.BlockSpec((tile, x.shape[1]), lambda p, j: ((1 - p) * j, 0)),
            pl.BlockSpec(w.shape, lambda p, j: (0, 0)),
            pl.BlockSpec((1, c), lambda p, j: (0, 0)),
            pl.BlockSpec((1, c), lambda p, j: (0, 0)),
            pl.BlockSpec((1, c), lambda p, j: (0, 0)),
        ],
        out_specs=pl.BlockSpec((tile, c), lambda p, j: (p * j, 0)),
        out_shape=jax.ShapeDtypeStruct((n, c), jnp.float32),
        scratch_shapes=[
            pltpu.VMEM((n, c), jnp.float32),
            pltpu.VMEM((2, c), jnp.float32),
        ],
    )(x, w, b[None, :], gamma[None, :], beta[None, :])


def _knn_topk_body(p1_ref, p2t_ref, i1_ref, i2_ref, i3_ref, wexp_ref):
    a = p1_ref[0]
    pt = p2t_ref[0]
    tq, n2 = a.shape[0], pt.shape[1]
    acc = jnp.zeros((tq, n2), jnp.float32)
    for d in range(3):
        t = a[:, d:d + 1] - pt[d:d + 1, :]
        acc = acc + t * t

    lane = lax.broadcasted_iota(jnp.int32, (tq, n2), 1)
    key = (lax.bitcast_convert_type(acc, jnp.int32) & _IMASK) | lane
    k1 = jnp.min(key, axis=1, keepdims=True)
    keyb = jnp.where(key == k1, _IMAX, key)
    k2 = jnp.min(keyb, axis=1, keepdims=True)
    keyc = jnp.where(keyb == k2, _IMAX, keyb)
    k3 = jnp.min(keyc, axis=1, keepdims=True)

    i1, i2, i3 = k1 & 2047, k2 & 2047, k3 & 2047
    d1 = lax.bitcast_convert_type(k1 - i1, jnp.float32)
    d2 = lax.bitcast_convert_type(k2 - i2, jnp.float32)
    d3 = lax.bitcast_convert_type(k3 - i3, jnp.float32)
    r1 = 1.0 / (d1 + 1e-8)
    r2 = 1.0 / (d2 + 1e-8)
    r3 = 1.0 / (d3 + 1e-8)
    inv = 1.0 / (r1 + r2 + r3)
    boff = pl.program_id(0) * n2
    i1_ref[0, 0] = jnp.transpose(i1 + boff, (1, 0))
    i2_ref[0, 0] = jnp.transpose(i2 + boff, (1, 0))
    i3_ref[0, 0] = jnp.transpose(i3 + boff, (1, 0))
    wexp_ref[0, 0] = jnp.concatenate(
        [jnp.broadcast_to(r1 * inv, (tq, 16)),
         jnp.broadcast_to(r2 * inv, (tq, 16)),
         jnp.broadcast_to(r3 * inv, (tq, 16))], axis=1)


def _knn_topk(p1, p2t, tile):
    b, n1, _ = p1.shape
    n2 = p2t.shape[2]
    nt = n1 // tile
    ispec = pl.BlockSpec((1, 1, 1, tile), lambda i, j: (i, j, 0, 0))
    ishape = jax.ShapeDtypeStruct((b, nt, 1, tile), jnp.int32)
    return pl.pallas_call(
        _knn_topk_body,
        grid=(b, nt),
        in_specs=[
            pl.BlockSpec((1, tile, 3), lambda i, j: (i, j, 0)),
            pl.BlockSpec((1, 3, n2), lambda i, j: (i, 0, 0)),
        ],
        out_specs=[
            ispec, ispec, ispec,
            pl.BlockSpec((1, 1, tile, 48), lambda i, j: (i, j, 0, 0)),
        ],
        out_shape=[
            ishape, ishape, ishape,
            jax.ShapeDtypeStruct((b, nt, tile, 48), jnp.float32),
        ],
    )(p1, p2t)


def _make_sc_interp(n_rows, c, n_workers, chunk):
    qpw = n_rows // n_workers
    n_chunks = qpw // chunk
    mesh = plsc.VectorSubcoreMesh(core_axis_name="c", subcore_axis_name="s")

    @functools.partial(
        pl.kernel, mesh=mesh,
        out_type=jax.ShapeDtypeStruct((n_rows, c), jnp.float32),
        scratch_types=[
            pltpu.VMEM((3, chunk), jnp.int32),
            pltpu.VMEM((3, chunk), jnp.int32),
            pltpu.VMEM((3, chunk), jnp.int32),
            pltpu.VMEM((3, chunk, 48), jnp.float32),
            pltpu.VMEM((3, chunk, c), jnp.float32),
            pltpu.VMEM((2, chunk, c), jnp.float32),
            pltpu.VMEM((2, chunk, c), jnp.float32),
            pltpu.VMEM((2, chunk, c), jnp.float32),
            pltpu.VMEM((2, chunk, c), jnp.float32),
        ] + [pltpu.SemaphoreType.DMA] * 7,
    )
    def sc_interp(z2_hbm, x1p_hbm, i1_hbm, i2_hbm, i3_hbm, wexp_hbm, out_hbm,
                  i1_v, i2_v, i3_v, w_v, x1p_v,
                  r1_v, r2_v, r3_v, out_v,
                  sl0, sl1, sl2, sg0, sg1, ss0, ss1):
        wid = lax.axis_index("s") * 2 + lax.axis_index("c")
        wbase = wid * qpw
        sls = (sl0, sl1, sl2)
        sgs = (sg0, sg1)
        sss = (ss0, ss1)
        lin_h = {}
        g_h = {}
        st_h = {}

        def issue_linear(ci):
            p = ci % 3
            sl = pl.ds(wbase + ci * chunk, chunk)
            lin_h[ci] = [
                pltpu.async_copy(i1_hbm.at[sl], i1_v.at[p], sls[p]),
                pltpu.async_copy(i2_hbm.at[sl], i2_v.at[p], sls[p]),
                pltpu.async_copy(i3_hbm.at[sl], i3_v.at[p], sls[p]),
                pltpu.async_copy(wexp_hbm.at[sl], w_v.at[p], sls[p]),
                pltpu.async_copy(x1p_hbm.at[sl], x1p_v.at[p], sls[p]),
            ]

        def issue_gathers(ci):
            p3, p2 = ci % 3, ci % 2
            for h in lin_h.pop(ci):
                h.wait()
            g_h[ci] = [
                pltpu.async_copy(z2_hbm.at[i1_v.at[p3]], r1_v.at[p2], sgs[p2]),
                pltpu.async_copy(z2_hbm.at[i2_v.at[p3]], r2_v.at[p2], sgs[p2]),
                pltpu.async_copy(z2_hbm.at[i3_v.at[p3]], r3_v.at[p2], sgs[p2]),
            ]

        issue_linear(0)
        issue_gathers(0)
        if n_chunks > 1:
            issue_linear(1)

        for ci in range(n_chunks):
            if ci + 2 < n_chunks:
                issue_linear(ci + 2)
            if ci + 1 < n_chunks:
                issue_gathers(ci + 1)
            p3, p2 = ci % 3, ci % 2
            for h in g_h.pop(ci):
                h.wait()
            if ci - 2 in st_h:
                st_h.pop(ci - 2).wait()

            def q_body(q, _):
                a1 = w_v[p3, q, 0:16]
                a2 = w_v[p3, q, 16:32]
                a3 = w_v[p3, q, 32:48]
                for cc in range(c // 16):
                    s = pl.ds(cc * 16, 16)
                    out_v[p2, q, s] = (
                        a1 * r1_v[p2, q, s] + a2 * r2_v[p2, q, s]
                        + a3 * r3_v[p2, q, s] + x1p_v[p3, q, s])
                return 0

            lax.fori_loop(0, chunk, q_body, 0)
            st_h[ci] = pltpu.async_copy(
                out_v.at[p2], out_hbm.at[pl.ds(wbase + ci * chunk, chunk)],
                sss[p2])
        for h in st_h.values():
            h.wait()

    return sc_interp


@jax.jit
def kernel(p1, x1, p2, x2, W1, b1, g1, be1, W2, b2, g2, be2):
    B, N1, _ = p1.shape
    N2 = p2.shape[1]
    C = W1.shape[0]

    z2 = _linear_bn_relu(x2.reshape(B * N2, -1), W2, b2, g2, be2, 1024)
    x1p = _linear_bn_relu(x1.reshape(B * N1, -1), W1, b1, g1, be1, 1024)

    p2t = jnp.transpose(p2, (0, 2, 1))
    tile = 512
    i1, i2, i3, wexp = _knn_topk(p1, p2t, tile)
    sc_interp = _make_sc_interp(B * N1, C, 32, 64)
    out = sc_interp(z2, x1p, i1.reshape(B * N1), i2.reshape(B * N1),
                    i3.reshape(B * N1), wexp.reshape(B * N1, 48))
    return out.reshape(B, N1, C)


# speedup vs baseline: 1.1170x; 1.0160x over previous
import functools

import jax
import jax.numpy as jnp
from jax import lax
from jax.experimental import pallas as pl
from jax.experimental.pallas import tpu as pltpu
from jax.experimental.pallas import tpu_sc as plsc

_IMASK = -2048
_IMAX = 2147483647


def _linear_bn_relu_body(n, eps, x_ref, wt_ref, b_ref, g_ref, be_ref,
                         z_ref, y_s, acc_s):
    p = pl.program_id(0)
    j = pl.program_id(1)
    tile = x_ref.shape[0]

    @pl.when((p == 0) & (j == 0))
    def _():
        acc_s[...] = jnp.zeros_like(acc_s)

    @pl.when(p == 0)
    def _():
        y = lax.dot_general(
            x_ref[...], wt_ref[...], (((1,), (1,)), ((), ())),
            preferred_element_type=jnp.float32) + b_ref[...]
        y_s[pl.ds(j * tile, tile), :] = y
        s = jnp.sum(y, axis=0, keepdims=True)
        ss = jnp.sum(y * y, axis=0, keepdims=True)
        acc_s[...] += jnp.concatenate([s, ss], axis=0)

    @pl.when(p == 1)
    def _():
        mean = acc_s[0:1, :] / n
        var = acc_s[1:2, :] / n - mean * mean
        sc = g_ref[...] * lax.rsqrt(var + eps)
        sh = be_ref[...] - mean * sc
        z_ref[...] = jnp.maximum(
            y_s[pl.ds(j * tile, tile), :] * sc + sh, 0.0)


def _linear_bn_relu(x, w, b, gamma, beta, tile, eps=1e-5):
    n, _ = x.shape
    c = w.shape[0]
    body = functools.partial(_linear_bn_relu_body, float(n), eps)
    return pl.pallas_call(
        body,
        grid=(2, n // tile),
        in_specs=[
            pl.BlockSpec((tile, x.shape[1]), lambda p, j: ((1 - p) * j, 0)),
            pl.BlockSpec(w.shape, lambda p, j: (0, 0)),
            pl.BlockSpec((1, c), lambda p, j: (0, 0)),
            pl.BlockSpec((1, c), lambda p, j: (0, 0)),
            pl.BlockSpec((1, c), lambda p, j: (0, 0)),
        ],
        out_specs=pl.BlockSpec((tile, c), lambda p, j: (p * j, 0)),
        out_shape=jax.ShapeDtypeStruct((n, c), jnp.float32),
        scratch_shapes=[
            pltpu.VMEM((n, c), jnp.float32),
            pltpu.VMEM((2, c), jnp.float32),
        ],
    )(x, w, b[None, :], gamma[None, :], beta[None, :])


def _knn_topk_body(p1_ref, p2t_ref, i1_ref, i2_ref, i3_ref, wexp_ref):
    a = p1_ref[0]
    pt = p2t_ref[0]
    tq, n2 = a.shape[0], pt.shape[1]
    acc = jnp.zeros((tq, n2), jnp.float32)
    for d in range(3):
        t = a[:, d:d + 1] - pt[d:d + 1, :]
        acc = acc + t * t

    lane = lax.broadcasted_iota(jnp.int32, (tq, n2), 1)
    key = (lax.bitcast_convert_type(acc, jnp.int32) & _IMASK) | lane
    k1 = jnp.min(key, axis=1, keepdims=True)
    keyb = jnp.where(key == k1, _IMAX, key)
    k2 = jnp.min(keyb, axis=1, keepdims=True)
    keyc = jnp.where(keyb == k2, _IMAX, keyb)
    k3 = jnp.min(keyc, axis=1, keepdims=True)

    i1, i2, i3 = k1 & 2047, k2 & 2047, k3 & 2047
    d1 = lax.bitcast_convert_type(k1 - i1, jnp.float32)
    d2 = lax.bitcast_convert_type(k2 - i2, jnp.float32)
    d3 = lax.bitcast_convert_type(k3 - i3, jnp.float32)
    r1 = 1.0 / (d1 + 1e-8)
    r2 = 1.0 / (d2 + 1e-8)
    r3 = 1.0 / (d3 + 1e-8)
    inv = 1.0 / (r1 + r2 + r3)
    boff = pl.program_id(0) * n2
    i1_ref[0, 0] = jnp.transpose(i1 + boff, (1, 0))
    i2_ref[0, 0] = jnp.transpose(i2 + boff, (1, 0))
    i3_ref[0, 0] = jnp.transpose(i3 + boff, (1, 0))
    wexp_ref[0, 0] = jnp.concatenate(
        [jnp.broadcast_to(r1 * inv, (tq, 16)),
         jnp.broadcast_to(r2 * inv, (tq, 16)),
         jnp.broadcast_to(r3 * inv, (tq, 16))], axis=1)


def _knn_topk(p1, p2t, tile):
    b, n1, _ = p1.shape
    n2 = p2t.shape[2]
    nt = n1 // tile
    ispec = pl.BlockSpec((1, 1, 1, tile), lambda i, j: (i, j, 0, 0))
    ishape = jax.ShapeDtypeStruct((b, nt, 1, tile), jnp.int32)
    return pl.pallas_call(
        _knn_topk_body,
        grid=(b, nt),
        in_specs=[
            pl.BlockSpec((1, tile, 3), lambda i, j: (i, j, 0)),
            pl.BlockSpec((1, 3, n2), lambda i, j: (i, 0, 0)),
        ],
        out_specs=[
            ispec, ispec, ispec,
            pl.BlockSpec((1, 1, tile, 48), lambda i, j: (i, j, 0, 0)),
        ],
        out_shape=[
            ishape, ishape, ishape,
            jax.ShapeDtypeStruct((b, nt, tile, 48), jnp.float32),
        ],
    )(p1, p2t)


def _make_sc_interp(n_rows, c, n_workers, chunk):
    qpw = n_rows // n_workers
    n_chunks = qpw // chunk
    mesh = plsc.VectorSubcoreMesh(core_axis_name="c", subcore_axis_name="s")

    @functools.partial(
        pl.kernel, mesh=mesh,
        out_type=jax.ShapeDtypeStruct((n_rows, c), jnp.float32),
        scratch_types=[
            pltpu.VMEM((3, chunk), jnp.int32),
            pltpu.VMEM((3, chunk), jnp.int32),
            pltpu.VMEM((3, chunk), jnp.int32),
            pltpu.VMEM((3, chunk, 48), jnp.float32),
            pltpu.VMEM((3, chunk, c), jnp.float32),
            pltpu.VMEM((2, chunk, c), jnp.float32),
            pltpu.VMEM((2, chunk, c), jnp.float32),
            pltpu.VMEM((2, chunk, c), jnp.float32),
            pltpu.VMEM((2, chunk, c), jnp.float32),
        ] + [pltpu.SemaphoreType.DMA] * 7,
    )
    def sc_interp(z2_hbm, x1p_hbm, i1_hbm, i2_hbm, i3_hbm, wexp_hbm, out_hbm,
                  i1_v, i2_v, i3_v, w_v, x1p_v,
                  r1_v, r2_v, r3_v, out_v,
                  sl0, sl1, sl2, sg0, sg1, ss0, ss1):
        wid = lax.axis_index("s") * 2 + lax.axis_index("c")
        wbase = wid * qpw
        sls = (sl0, sl1, sl2)
        sgs = (sg0, sg1)
        sss = (ss0, ss1)
        lin_h = {}
        g_h = {}
        st_h = {}

        def issue_linear(ci):
            p = ci % 3
            sl = pl.ds(wbase + ci * chunk, chunk)
            lin_h[ci] = [
                pltpu.async_copy(i1_hbm.at[sl], i1_v.at[p], sls[p]),
                pltpu.async_copy(i2_hbm.at[sl], i2_v.at[p], sls[p]),
                pltpu.async_copy(i3_hbm.at[sl], i3_v.at[p], sls[p]),
                pltpu.async_copy(wexp_hbm.at[sl], w_v.at[p], sls[p]),
                pltpu.async_copy(x1p_hbm.at[sl], x1p_v.at[p], sls[p]),
            ]

        def issue_gathers(ci):
            p3, p2 = ci % 3, ci % 2
            for h in lin_h.pop(ci):
                h.wait()
            g_h[ci] = [
                pltpu.async_copy(z2_hbm.at[i1_v.at[p3]], r1_v.at[p2], sgs[p2]),
                pltpu.async_copy(z2_hbm.at[i2_v.at[p3]], r2_v.at[p2], sgs[p2]),
                pltpu.async_copy(z2_hbm.at[i3_v.at[p3]], r3_v.at[p2], sgs[p2]),
            ]

        issue_linear(0)
        issue_gathers(0)
        if n_chunks > 1:
            issue_linear(1)

        for ci in range(n_chunks):
            if ci + 2 < n_chunks:
                issue_linear(ci + 2)
            if ci + 1 < n_chunks:
                issue_gathers(ci + 1)
            p3, p2 = ci % 3, ci % 2
            for h in g_h.pop(ci):
                h.wait()
            if ci - 2 in st_h:
                st_h.pop(ci - 2).wait()

            def q_body(q, _):
                a1 = w_v[p3, q, 0:16]
                a2 = w_v[p3, q, 16:32]
                a3 = w_v[p3, q, 32:48]
                for cc in range(c // 16):
                    s = pl.ds(cc * 16, 16)
                    out_v[p2, q, s] = (
                        a1 * r1_v[p2, q, s] + a2 * r2_v[p2, q, s]
                        + a3 * r3_v[p2, q, s] + x1p_v[p3, q, s])
                return 0

            lax.fori_loop(0, chunk, q_body, 0)
            st_h[ci] = pltpu.async_copy(
                out_v.at[p2], out_hbm.at[pl.ds(wbase + ci * chunk, chunk)],
                sss[p2])
        for h in st_h.values():
            h.wait()

    return sc_interp


@jax.jit
def kernel(p1, x1, p2, x2, W1, b1, g1, be1, W2, b2, g2, be2):
    B, N1, _ = p1.shape
    N2 = p2.shape[1]
    C = W1.shape[0]

    z2 = _linear_bn_relu(x2.reshape(B * N2, -1), W2, b2, g2, be2, 1024)
    x1p = _linear_bn_relu(x1.reshape(B * N1, -1), W1, b1, g1, be1, 1024)

    p2t = jnp.transpose(p2, (0, 2, 1))
    tile = 1024
    i1, i2, i3, wexp = _knn_topk(p1, p2t, tile)
    sc_interp = _make_sc_interp(B * N1, C, 32, 64)
    out = sc_interp(z2, x1p, i1.reshape(B * N1), i2.reshape(B * N1),
                    i3.reshape(B * N1), wexp.reshape(B * N1, 48))
    return out.reshape(B, N1, C)
